# baseline (speedup 1.0000x reference)
import functools

import jax
import jax.numpy as jnp
from jax import lax
from jax.experimental import pallas as pl
from jax.experimental.pallas import tpu as pltpu
from jax.experimental.pallas import tpu_sc as plsc

NC = 2
NS = 16
NW = NC * NS

BATCH = 16384
DIM = 32
PACK = 128 // DIM
NROWS4 = 1000000 // PACK
CHUNK = 128
CHUNKS_PER_W = BATCH // (NW * CHUNK)


def _sc_gather4(u2, i2, ug4, ig4, um4, im4):
    out_t = jax.ShapeDtypeStruct((BATCH, 128), jnp.float32)
    mesh = plsc.VectorSubcoreMesh(core_axis_name="c", subcore_axis_name="s")

    @functools.partial(
        pl.kernel,
        out_type=(out_t, out_t, out_t, out_t),
        mesh=mesh,
        scratch_types=[
            pltpu.VMEM((CHUNKS_PER_W, 128), jnp.int32),
            pltpu.VMEM((CHUNKS_PER_W, 128), jnp.int32),
            pltpu.VMEM((CHUNK, 128), jnp.float32),
            pltpu.VMEM((CHUNK, 128), jnp.float32),
            pltpu.VMEM((CHUNK, 128), jnp.float32),
            pltpu.VMEM((CHUNK, 128), jnp.float32),
            pltpu.SemaphoreType.DMA,
            pltpu.SemaphoreType.DMA,
            pltpu.SemaphoreType.DMA,
            pltpu.SemaphoreType.DMA,
        ],
    )
    def k(ug_hbm, ig_hbm, um_hbm, im_hbm, u_hbm, i_hbm,
          oug, oig, oum, oim,
          uidx, iidx, r_ug, r_ig, r_um, r_im, s1, s2, s3, s4):
        wid = lax.axis_index("s") * NC + lax.axis_index("c")
        row0 = wid * CHUNKS_PER_W
        pltpu.sync_copy(u_hbm.at[pl.ds(row0, CHUNKS_PER_W)], uidx)
        pltpu.sync_copy(i_hbm.at[pl.ds(row0, CHUNKS_PER_W)], iidx)
        for j in range(CHUNKS_PER_W):
            base = (row0 + j) * CHUNK
            descs = (
                pltpu.async_copy(ug_hbm.at[uidx.at[j]], r_ug, s1),
                pltpu.async_copy(ig_hbm.at[iidx.at[j]], r_ig, s2),
                pltpu.async_copy(um_hbm.at[uidx.at[j]], r_um, s3),
                pltpu.async_copy(im_hbm.at[iidx.at[j]], r_im, s4),
            )
            for d, buf, out in ((descs[0], r_ug, oug), (descs[1], r_ig, oig),
                                (descs[2], r_um, oum), (descs[3], r_im, oim)):
                d.wait()
                pltpu.sync_copy(buf, out.at[pl.ds(base, CHUNK)])

    return k(ug4, ig4, um4, im4, u2, i2)


BLK = 2048


def _sel(x, m):
    r = jnp.where(m == 0, x[:, 0 * DIM:1 * DIM], x[:, 1 * DIM:2 * DIM])
    r = jnp.where(m == 2, x[:, 2 * DIM:3 * DIM], r)
    return jnp.where(m == 3, x[:, 3 * DIM:4 * DIM], r)


def _tc_body(us_ref, is_ref, ug_ref, ig_ref, um_ref, im_ref,
             w0u_ref, w0i_ref, b0_ref, w1_ref, b1_ref, w2_ref, b2_ref,
             whg_ref, whh_ref, bh_ref, o_ref):
    f32 = jnp.float32
    mu = us_ref[...]
    mi = is_ref[...]
    ug = _sel(ug_ref[...], mu)
    ig = _sel(ig_ref[...], mi)
    um = _sel(um_ref[...], mu)
    im = _sel(im_ref[...], mi)
    h = jnp.dot(um, w0u_ref[...], preferred_element_type=f32)
    h = h + jnp.dot(im, w0i_ref[...], preferred_element_type=f32)
    h = jnp.maximum(h + b0_ref[...], 0.0)
    h = jnp.maximum(jnp.dot(h, w1_ref[...], preferred_element_type=f32)
                    + b1_ref[...], 0.0)
    h = jnp.maximum(jnp.dot(h, w2_ref[...], preferred_element_type=f32)
                    + b2_ref[...], 0.0)
    gmf = ug * ig
    o_ref[...] = (jnp.dot(gmf, whg_ref[...], preferred_element_type=f32)
                  + jnp.dot(h, whh_ref[...], preferred_element_type=f32)
                  + bh_ref[...])


def _tc_dense(us, is_, ug, ig, um, im, W0, b0, W1, b1, W2, b2, Wh, bh):
    w0u = W0[:, :DIM].T
    w0i = W0[:, DIM:].T
    w1 = W1.T
    w2 = W2.T
    whg = Wh[:, :DIM].T
    whh = Wh[:, DIM:].T
    b0r = b0.reshape(1, -1)
    b1r = b1.reshape(1, -1)
    b2r = b2.reshape(1, -1)
    bhr = bh.reshape(1, 1)

    n_blk = BATCH // BLK
    row_spec = pl.BlockSpec((BLK, 128), lambda b: (b, 0))
    sel_spec = pl.BlockSpec((BLK, 1), lambda b: (b, 0))

    def w_spec(shape):
        return pl.BlockSpec(shape, lambda b: (0, 0))

    out = pl.pallas_call(
        _tc_body,
        grid=(n_blk,),
        in_specs=[
            sel_spec, sel_spec,
            row_spec, row_spec, row_spec, row_spec,
            w_spec(w0u.shape), w_spec(w0i.shape), w_spec(b0r.shape),
            w_spec(w1.shape), w_spec(b1r.shape),
            w_spec(w2.shape), w_spec(b2r.shape),
            w_spec(whg.shape), w_spec(whh.shape), w_spec(bhr.shape),
        ],
        out_specs=pl.BlockSpec((BLK, 1), lambda b: (b, 0)),
        out_shape=jax.ShapeDtypeStruct((BATCH, 1), jnp.float32),
    )(us, is_, ug, ig, um, im, w0u, w0i, b0r, w1, b1r, w2, b2r,
      whg, whh, bhr)
    return out[:, 0]


def kernel(u, i, user_gmf, item_gmf, user_mlp, item_mlp,
           W0, b0, W1, b1, W2, b2, Wh, bh):
    u2 = (u >> 2).reshape(BATCH // 128, 128)
    i2 = (i >> 2).reshape(BATCH // 128, 128)
    ug, ig, um, im = _sc_gather4(
        u2, i2,
        user_gmf.reshape(NROWS4, 128), item_gmf.reshape(NROWS4, 128),
        user_mlp.reshape(NROWS4, 128), item_mlp.reshape(NROWS4, 128))
    us = (u & 3).reshape(BATCH, 1)
    is_ = (i & 3).reshape(BATCH, 1)
    return _tc_dense(us, is_, ug, ig, um, im,
                     W0, b0, W1, b1, W2, b2, Wh, bh)
